# Initial kernel scaffold; baseline (speedup 1.0000x reference)
import jax
import jax.numpy as jnp
from jax.experimental import pallas as pl


def kernel(input, embed, pos_embed_weight, ffn_w, ffn_b, ln_w, ln_b):
    raise NotImplementedError("write your pallas kernel here")


# speedup vs baseline: 1.0517x; 1.0517x over previous
import functools

import jax
import jax.numpy as jnp
from jax import lax
from jax.experimental import pallas as pl
from jax.experimental.pallas import tpu as pltpu
from jax.experimental.pallas import tpu_sc as plsc

V = 1000000
E = 64
H = 4
C = 3
B = 1024
T = 50
EPS = 1e-05

NC = 2
NS = 16
NW = NC * NS
BT = B * T
RPW = BT // NW
CH = 80
NCH = RPW // CH


def _sc_gather(idx3, table):
    mesh = plsc.VectorSubcoreMesh(core_axis_name="c", subcore_axis_name="s")

    @functools.partial(
        pl.kernel,
        mesh=mesh,
        out_type=jax.ShapeDtypeStruct((BT, E), jnp.float32),
        scratch_types=[
            pltpu.VMEM((NCH, CH), jnp.int32),
            pltpu.VMEM((RPW, E), jnp.float32),
            pltpu.SemaphoreType.DMA,
        ],
        compiler_params=pltpu.CompilerParams(use_tc_tiling_on_sc=False),
    )
    def k(idx_hbm, table_hbm, out_hbm, idx_v, rows_v, sem):
        wid = lax.axis_index("s") * NC + lax.axis_index("c")
        pltpu.sync_copy(idx_hbm.at[wid], idx_v)
        copies = []
        for j in range(NCH):
            copies.append(
                pltpu.async_copy(
                    table_hbm.at[idx_v.at[j]],
                    rows_v.at[pl.ds(j * CH, CH)],
                    sem,
                )
            )
        for cp in copies:
            cp.wait()
        pltpu.sync_copy(rows_v, out_hbm.at[pl.ds(wid * RPW, RPW)])

    return k(idx3, table)


def _tc_body(g_ref, m_ref, wt_ref, b_ref, lnw_ref, lnb_ref, o_ref, *, blk):
    g = g_ref[...]
    m = m_ref[...]
    t = lax.broadcasted_iota(jnp.int32, (blk, 1), 0) % T
    d0 = jnp.sum(g * m[0:1], axis=-1, keepdims=True)
    d1 = jnp.sum(g * m[1:2], axis=-1, keepdims=True)
    d2 = jnp.sum(g * m[2:3], axis=-1, keepdims=True)
    s0 = d0 * g
    s1 = d1 * g
    s2 = d2 * g
    sh1 = jnp.concatenate([jnp.zeros((1, E), g.dtype), s1[:-1]], axis=0)
    sh2 = jnp.concatenate([jnp.zeros((2, E), g.dtype), s0[:-2]], axis=0)
    out = s2 + jnp.where(t >= 1, sh1, 0.0) + jnp.where(t >= 2, sh2, 0.0)
    out = out * (1.0 / (H * C))
    y = jnp.dot(out, wt_ref[...], preferred_element_type=jnp.float32)
    y = y + b_ref[...]
    mean = jnp.mean(y, axis=-1, keepdims=True)
    yc = y - mean
    var = jnp.mean(yc * yc, axis=-1, keepdims=True)
    yn = yc * lax.rsqrt(var + EPS) * lnw_ref[...] + lnb_ref[...]
    o_ref[...] = yn * jax.nn.sigmoid(yn)


def _tc_compute(g, m, wt, bias, lnw, lnb):
    blk = 6400
    grid = BT // blk
    return pl.pallas_call(
        functools.partial(_tc_body, blk=blk),
        grid=(grid,),
        in_specs=[
            pl.BlockSpec((blk, E), lambda i: (i, 0)),
            pl.BlockSpec((8, E), lambda i: (0, 0)),
            pl.BlockSpec((E, E), lambda i: (0, 0)),
            pl.BlockSpec((1, E), lambda i: (0, 0)),
            pl.BlockSpec((1, E), lambda i: (0, 0)),
            pl.BlockSpec((1, E), lambda i: (0, 0)),
        ],
        out_specs=pl.BlockSpec((blk, E), lambda i: (i, 0)),
        out_shape=jax.ShapeDtypeStruct((BT, E), jnp.float32),
    )(g, m, wt, bias, lnw, lnb)


def kernel(input, embed, pos_embed_weight, ffn_w, ffn_b, ln_w, ln_b):
    idx3 = input.astype(jnp.int32).reshape(NW, NCH, CH)
    g = _sc_gather(idx3, embed)
    m = pos_embed_weight.reshape(H, E, C).transpose(0, 2, 1).sum(axis=0)
    m = jnp.concatenate([m, jnp.zeros((8 - C, E), m.dtype)], axis=0)
    out = _tc_compute(
        g,
        m,
        ffn_w.T,
        ffn_b.reshape(1, E),
        ln_w.reshape(1, E),
        ln_b.reshape(1, E),
    )
    return out.reshape(B, T, E)
